# baseline (speedup 1.0000x reference)
import functools

import jax
import jax.numpy as jnp
from jax import lax
from jax.experimental import pallas as pl
from jax.experimental.pallas import tpu as pltpu
from jax.experimental.pallas import tpu_sc as plsc


def _make_gather(N, V, D, NC, NS, CHUNK):
    NW = NC * NS
    per_w = N // NW
    nch = per_w // CHUNK
    assert per_w % CHUNK == 0 and nch % 2 == 0 and per_w % 8 == 0

    mesh = plsc.VectorSubcoreMesh(core_axis_name="c", subcore_axis_name="s")

    @functools.partial(
        pl.kernel,
        out_type=jax.ShapeDtypeStruct((N, D), jnp.float32),
        mesh=mesh,
        compiler_params=pltpu.CompilerParams(use_tc_tiling_on_sc=False),
        scratch_types=[
            pltpu.VMEM((per_w,), jnp.int32),
            pltpu.VMEM((CHUNK, D), jnp.float32),
            pltpu.VMEM((CHUNK, D), jnp.float32),
            pltpu.SemaphoreType.DMA,
            pltpu.SemaphoreType.DMA,
        ],
    )
    def gather_kernel(ids_hbm, table_hbm, out_hbm, idx_v, buf0, buf1, sem0, sem1):
        wid = lax.axis_index("s") * NC + lax.axis_index("c")
        base = pl.multiple_of(wid * per_w, 8)
        pltpu.sync_copy(ids_hbm.at[pl.ds(base, per_w)], idx_v)

        def idx_slice(c):
            return idx_v.at[pl.ds(pl.multiple_of(c * CHUNK, 8), CHUNK)]

        def gather_cp(c, buf, sem):
            return pltpu.make_async_copy(table_hbm.at[idx_slice(c)], buf, sem)

        def out_write(c, buf):
            start = pl.multiple_of(base + c * CHUNK, 8)
            pltpu.sync_copy(buf, out_hbm.at[pl.ds(start, CHUNK)])

        gather_cp(0, buf0, sem0).start()
        gather_cp(1, buf1, sem1).start()

        def body(g, carry):
            c = 2 * g
            gather_cp(c, buf0, sem0).wait()
            out_write(c, buf0)
            gather_cp(c + 2, buf0, sem0).start()
            gather_cp(c + 1, buf1, sem1).wait()
            out_write(c + 1, buf1)
            gather_cp(c + 3, buf1, sem1).start()
            return carry

        lax.fori_loop(0, nch // 2 - 1, body, 0, unroll=False)

        c_last = nch - 2
        gather_cp(c_last, buf0, sem0).wait()
        out_write(c_last, buf0)
        gather_cp(c_last + 1, buf1, sem1).wait()
        out_write(c_last + 1, buf1)

    return gather_kernel


def _mask_body(ids_ref, m_ref):
    m_ref[...] = ids_ref[...] != 0


def kernel(ids, table):
    B, T = ids.shape
    V, D = table.shape
    N = B * T

    info = plsc.get_sparse_core_info()
    NC, NS = info.num_cores, info.num_subcores

    gather = _make_gather(N, V, D, NC, NS, CHUNK=512)
    flat_ids = ids.reshape(N)
    tokens = gather(flat_ids, table).reshape(B, T, D)

    mask = pl.pallas_call(
        _mask_body,
        out_shape=jax.ShapeDtypeStruct((B, T), jnp.bool_),
    )(ids)
    return tokens, mask


# speedup vs baseline: 1.0003x; 1.0003x over previous
import functools

import jax
import jax.numpy as jnp
from jax import lax
from jax.experimental import pallas as pl
from jax.experimental.pallas import tpu as pltpu
from jax.experimental.pallas import tpu_sc as plsc


def _make_gather(N, V, D, NC, NS, CHUNK):
    NW = NC * NS
    per_w = N // NW
    nch = per_w // CHUNK
    assert per_w % CHUNK == 0 and nch % 2 == 0 and per_w % 8 == 0

    mesh = plsc.VectorSubcoreMesh(core_axis_name="c", subcore_axis_name="s")

    @functools.partial(
        pl.kernel,
        out_type=jax.ShapeDtypeStruct((N, D), jnp.float32),
        mesh=mesh,
        compiler_params=pltpu.CompilerParams(
            use_tc_tiling_on_sc=False, skip_device_barrier=True
        ),
        scratch_types=[
            pltpu.VMEM((per_w,), jnp.int32),
            pltpu.VMEM((CHUNK, D), jnp.float32),
            pltpu.VMEM((CHUNK, D), jnp.float32),
            pltpu.SemaphoreType.DMA,
            pltpu.SemaphoreType.DMA,
        ],
    )
    def gather_kernel(ids_hbm, table_hbm, out_hbm, idx_v, buf0, buf1, sem0, sem1):
        wid = lax.axis_index("s") * NC + lax.axis_index("c")
        base = pl.multiple_of(wid * per_w, 8)
        pltpu.sync_copy(ids_hbm.at[pl.ds(base, per_w)], idx_v)

        def idx_slice(c):
            return idx_v.at[pl.ds(pl.multiple_of(c * CHUNK, 8), CHUNK)]

        def gather_cp(c, buf, sem):
            return pltpu.make_async_copy(table_hbm.at[idx_slice(c)], buf, sem)

        def out_write(c, buf):
            start = pl.multiple_of(base + c * CHUNK, 8)
            pltpu.sync_copy(buf, out_hbm.at[pl.ds(start, CHUNK)])

        gather_cp(0, buf0, sem0).start()
        gather_cp(1, buf1, sem1).start()

        def body(g, carry):
            c = 2 * g
            gather_cp(c, buf0, sem0).wait()
            out_write(c, buf0)
            gather_cp(c + 2, buf0, sem0).start()
            gather_cp(c + 1, buf1, sem1).wait()
            out_write(c + 1, buf1)
            gather_cp(c + 3, buf1, sem1).start()
            return carry

        lax.fori_loop(0, nch // 2 - 1, body, 0, unroll=False)

        c_last = nch - 2
        gather_cp(c_last, buf0, sem0).wait()
        out_write(c_last, buf0)
        gather_cp(c_last + 1, buf1, sem1).wait()
        out_write(c_last + 1, buf1)

    return gather_kernel


def _mask_body(ids_ref, m_ref):
    m_ref[...] = ids_ref[...] != 0


def kernel(ids, table):
    B, T = ids.shape
    V, D = table.shape
    N = B * T

    info = plsc.get_sparse_core_info()
    NC, NS = info.num_cores, info.num_subcores

    gather = _make_gather(N, V, D, NC, NS, CHUNK=512)
    flat_ids = ids.reshape(N)
    tokens = gather(flat_ids, table).reshape(B, T, D)

    mask = pl.pallas_call(
        _mask_body,
        out_shape=jax.ShapeDtypeStruct((B, T), jnp.bool_),
    )(ids)
    return tokens, mask


# speedup vs baseline: 1.0022x; 1.0019x over previous
import functools

import jax
import jax.numpy as jnp
from jax import lax
from jax.experimental import pallas as pl
from jax.experimental.pallas import tpu as pltpu
from jax.experimental.pallas import tpu_sc as plsc


def _make_gather(B, T, V, D, NC, NS):
    NW = NC * NS
    rows_per_w = B // NW
    assert B % NW == 0 and rows_per_w % 4 == 0

    mesh = plsc.VectorSubcoreMesh(core_axis_name="c", subcore_axis_name="s")

    @functools.partial(
        pl.kernel,
        out_type=jax.ShapeDtypeStruct((B, T, D), jnp.float32),
        mesh=mesh,
        compiler_params=pltpu.CompilerParams(
            use_tc_tiling_on_sc=False, skip_device_barrier=True
        ),
        scratch_types=[
            pltpu.VMEM((rows_per_w, T), jnp.int32),
            pltpu.VMEM((T, D), jnp.float32),
            pltpu.VMEM((T, D), jnp.float32),
            pltpu.VMEM((T, D), jnp.float32),
            pltpu.VMEM((T, D), jnp.float32),
            pltpu.SemaphoreType.DMA,
            pltpu.SemaphoreType.DMA,
            pltpu.SemaphoreType.DMA,
            pltpu.SemaphoreType.DMA,
            pltpu.SemaphoreType.DMA,
            pltpu.SemaphoreType.DMA,
            pltpu.SemaphoreType.DMA,
            pltpu.SemaphoreType.DMA,
        ],
    )
    def gather_kernel(ids_hbm, table_hbm, out_hbm, idx_v, b0, b1, b2, b3,
                      g0, g1, g2, g3, w0, w1, w2, w3):
        bufs = (b0, b1, b2, b3)
        gsems = (g0, g1, g2, g3)
        wsems = (w0, w1, w2, w3)
        wid = lax.axis_index("s") * NC + lax.axis_index("c")
        base = wid * rows_per_w
        pltpu.sync_copy(ids_hbm.at[pl.ds(base, rows_per_w)], idx_v)

        def gather_cp(r, s):
            return pltpu.make_async_copy(
                table_hbm.at[idx_v.at[r]], bufs[s], gsems[s])

        def write_cp(r, s):
            return pltpu.make_async_copy(bufs[s], out_hbm.at[base + r], wsems[s])

        gather_cp(0, 0).start()
        gather_cp(1, 1).start()
        for s in range(4):
            gather_cp(s, s).wait()
            write_cp(s, s).start()
            if s >= 2:
                write_cp(s - 2, s - 2).wait()
            gather_cp(s + 2, (s + 2) % 4).start()

        def body(g, carry):
            r0 = 4 * g
            for s in range(4):
                r = r0 + s
                gather_cp(r, s).wait()
                write_cp(r, s).start()
                write_cp(r - 2, (s + 2) % 4).wait()
                gather_cp(r + 2, (s + 2) % 4).start()
            return carry

        lax.fori_loop(1, rows_per_w // 4 - 1, body, 0, unroll=False)

        rl = rows_per_w - 4
        for s in range(2):
            gather_cp(rl + s, s).wait()
            write_cp(rl + s, s).start()
            write_cp(rl + s - 2, (s + 2) % 4).wait()
            gather_cp(rl + s + 2, (s + 2) % 4).start()
        for s in range(2, 4):
            gather_cp(rl + s, s).wait()
            write_cp(rl + s, s).start()
        for s in range(4):
            write_cp(rl + s, s).wait()

    return gather_kernel


def _mask_body(ids_ref, m_ref):
    m_ref[...] = ids_ref[...] != 0


def kernel(ids, table):
    B, T = ids.shape
    V, D = table.shape

    info = plsc.get_sparse_core_info()
    NC, NS = info.num_cores, info.num_subcores

    gather = _make_gather(B, T, V, D, NC, NS)
    tokens = gather(ids, table)

    mask = pl.pallas_call(
        _mask_body,
        out_shape=jax.ShapeDtypeStruct((B, T), jnp.bool_),
    )(ids)
    return tokens, mask


# speedup vs baseline: 1.0424x; 1.0401x over previous
import functools

import jax
import jax.numpy as jnp
from jax import lax
from jax.experimental import pallas as pl
from jax.experimental.pallas import tpu as pltpu
from jax.experimental.pallas import tpu_sc as plsc


def _make_gather(B, T, V, D, NC, NS):
    NW = NC * NS
    rows_per_w = B // NW
    assert B % NW == 0 and rows_per_w % 4 == 0

    mesh = plsc.VectorSubcoreMesh(core_axis_name="c", subcore_axis_name="s")

    @functools.partial(
        pl.kernel,
        out_type=jax.ShapeDtypeStruct((B, T, D), jnp.float32),
        mesh=mesh,
        compiler_params=pltpu.CompilerParams(
            use_tc_tiling_on_sc=False, skip_device_barrier=True
        ),
        scratch_types=[
            pltpu.VMEM((rows_per_w, T), jnp.int32),
            pltpu.VMEM((T, D), jnp.float32),
            pltpu.VMEM((T, D), jnp.float32),
            pltpu.VMEM((T, D), jnp.float32),
            pltpu.VMEM((T, D), jnp.float32),
            pltpu.SemaphoreType.DMA,
            pltpu.SemaphoreType.DMA,
            pltpu.SemaphoreType.DMA,
            pltpu.SemaphoreType.DMA,
            pltpu.SemaphoreType.DMA,
            pltpu.SemaphoreType.DMA,
            pltpu.SemaphoreType.DMA,
            pltpu.SemaphoreType.DMA,
        ],
    )
    def gather_kernel(ids_hbm, table_hbm, out_hbm, idx_v, b0, b1, b2, b3,
                      g0, g1, g2, g3, w0, w1, w2, w3):
        bufs = (b0, b1, b2, b3)
        gsems = (g0, g1, g2, g3)
        wsems = (w0, w1, w2, w3)
        wid = lax.axis_index("s") * NC + lax.axis_index("c")
        base = wid * rows_per_w
        pltpu.sync_copy(ids_hbm.at[pl.ds(base, rows_per_w)], idx_v)

        def gather_cp(r, s):
            return pltpu.make_async_copy(
                table_hbm.at[idx_v.at[r]], bufs[s], gsems[s])

        def write_cp(r, s):
            return pltpu.make_async_copy(bufs[s], out_hbm.at[base + r], wsems[s])

        gather_cp(0, 0).start()
        gather_cp(1, 1).start()
        for s in range(4):
            gather_cp(s, s).wait()
            write_cp(s, s).start()
            if s >= 2:
                write_cp(s - 2, s - 2).wait()
            gather_cp(s + 2, (s + 2) % 4).start()

        def body(g, carry):
            r0 = 4 * g
            for s in range(4):
                r = r0 + s
                gather_cp(r, s).wait()
                write_cp(r, s).start()
                write_cp(r - 2, (s + 2) % 4).wait()
                gather_cp(r + 2, (s + 2) % 4).start()
            return carry

        lax.fori_loop(1, rows_per_w // 4 - 1, body, 0, unroll=False)

        rl = rows_per_w - 4
        for s in range(2):
            gather_cp(rl + s, s).wait()
            write_cp(rl + s, s).start()
            write_cp(rl + s - 2, (s + 2) % 4).wait()
            gather_cp(rl + s + 2, (s + 2) % 4).start()
        for s in range(2, 4):
            gather_cp(rl + s, s).wait()
            write_cp(rl + s, s).start()
        for s in range(4):
            write_cp(rl + s, s).wait()

    return gather_kernel


def _mask_body(ids_ref, m_ref):
    m_ref[...] = ids_ref[...] != 0


def _pack_body(tT_ref, o_ref):
    x = tT_ref[...]
    xt = x.T
    y = xt.reshape(xt.shape[0] // 2, 2, xt.shape[1])
    o_ref[...] = jnp.concatenate([y[:, 0, :], y[:, 1, :]], axis=1)


def _pack_table(tableT, W=2048):
    D, V = tableT.shape
    nb = (V + W - 1) // W
    rows = W * D // 128
    return pl.pallas_call(
        _pack_body,
        grid=(nb,),
        in_specs=[pl.BlockSpec((D, W), lambda i: (0, i))],
        out_specs=pl.BlockSpec((rows, 128), lambda i: (i, 0)),
        out_shape=jax.ShapeDtypeStruct((V * D // 128, 128), jnp.float32),
    )(tableT)


def kernel(ids, table):
    B, T = ids.shape
    V, D = table.shape

    info = plsc.get_sparse_core_info()
    NC, NS = info.num_cores, info.num_subcores

    packed = _pack_table(table.T)
    table_lin = packed.reshape(V, D)

    gather = _make_gather(B, T, V, D, NC, NS)
    tokens = gather(ids, table_lin)

    mask = pl.pallas_call(
        _mask_body,
        out_shape=jax.ShapeDtypeStruct((B, T), jnp.bool_),
    )(ids)
    return tokens, mask


# speedup vs baseline: 1.4244x; 1.3665x over previous
import functools

import jax
import jax.numpy as jnp
from jax import lax
from jax.experimental import pallas as pl
from jax.experimental.pallas import tpu as pltpu
from jax.experimental.pallas import tpu_sc as plsc


def _eye(n):
    a = lax.broadcasted_iota(jnp.int32, (n, n), 0)
    b = lax.broadcasted_iota(jnp.int32, (n, n), 1)
    return (a == b).astype(jnp.float32)


def _mxu_t(x):
    return lax.dot_general(
        x, _eye(x.shape[0]), (((0,), (0,)), ((), ())),
        preferred_element_type=jnp.float32)


def _pack_body(tT_ref, o_ref):
    xt = _mxu_t(tT_ref[...])
    y = xt.reshape(xt.shape[0] // 2, 2, xt.shape[1])
    o_ref[...] = jnp.concatenate([y[:, 0, :], y[:, 1, :]], axis=1)


def _pack_table(tableT, W=2048):
    D, V = tableT.shape
    nb = (V + W - 1) // W
    return pl.pallas_call(
        _pack_body,
        grid=(nb,),
        in_specs=[pl.BlockSpec((D, W), lambda i: (0, i))],
        out_specs=pl.BlockSpec((W * D // 128, 128), lambda i: (i, 0)),
        out_shape=jax.ShapeDtypeStruct((V * D // 128, 128), jnp.float32),
    )(tableT)


def _make_gather(B, T, V, D, NC, NS):
    NW = NC * NS
    bpw = B // NW
    npr = T // 2
    groups = npr // 2
    gr = 4 * bpw
    nvec = gr // 16
    assert B % NW == 0 and T % 4 == 0 and groups % 2 == 0 and bpw == 128

    mesh = plsc.VectorSubcoreMesh(core_axis_name="c", subcore_axis_name="s")

    @functools.partial(
        pl.kernel,
        out_type=jax.ShapeDtypeStruct((B * T, D), jnp.float32),
        mesh=mesh,
        compiler_params=pltpu.CompilerParams(
            use_tc_tiling_on_sc=False, skip_device_barrier=True,
            needs_layout_passes=False,
        ),
        scratch_types=[
            pltpu.VMEM((bpw * T,), jnp.int32),
            pltpu.VMEM((gr,), jnp.int32),
            pltpu.VMEM((gr,), jnp.int32),
            pltpu.VMEM((gr,), jnp.int32),
            pltpu.VMEM((gr, D), jnp.float32),
            pltpu.VMEM((gr, D), jnp.float32),
            pltpu.SemaphoreType.DMA,
            pltpu.SemaphoreType.DMA,
        ],
    )
    def gather_kernel(ids_hbm, table_hbm, out_hbm, idx_v, off_v,
                      ig0, ig1, b0, b1, g0, g1):
        igs = (ig0, ig1)
        bufs = (b0, b1)
        gsems = (g0, g1)
        wid = lax.axis_index("s") * NC + lax.axis_index("c")
        wrow = wid * bpw * T
        pltpu.sync_copy(ids_hbm.at[pl.ds(pl.multiple_of(wrow, 8), bpw * T)],
                        idx_v)

        for m in range(nvec):
            k = lax.broadcasted_iota(jnp.int32, (16,), 0) + (m * 16)
            jl = lax.shift_right_logical(k, 8)
            b = lax.shift_right_logical(k & (2 * bpw - 1), 1)
            h = k & 1
            off_v[pl.ds(m * 16, 16)] = b * T + 2 * jl + h

        def build_idx(g, s):
            tadd = 4 * g
            for m in range(nvec):
                ov = off_v[pl.ds(m * 16, 16)] + tadd
                igs[s][pl.ds(m * 16, 16)] = plsc.load_gather(idx_v, [ov])

        def gather_cp(s):
            return pltpu.make_async_copy(
                table_hbm.at[igs[s]], bufs[s], gsems[s])

        def out_write(g, s):
            start = pl.multiple_of(wrow + g * gr, 8)
            pltpu.sync_copy(bufs[s], out_hbm.at[pl.ds(start, gr)])

        build_idx(0, 0)
        gather_cp(0).start()
        build_idx(1, 1)
        gather_cp(1).start()

        def body(m, carry):
            g = 2 * m
            gather_cp(0).wait()
            out_write(g, 0)
            build_idx(g + 2, 0)
            gather_cp(0).start()
            gather_cp(1).wait()
            out_write(g + 1, 1)
            build_idx(g + 3, 1)
            gather_cp(1).start()
            return carry

        lax.fori_loop(0, groups // 2 - 1, body, 0, unroll=False)

        gather_cp(0).wait()
        out_write(groups - 2, 0)
        gather_cp(1).wait()
        out_write(groups - 1, 1)

    return gather_kernel


def _unpack_body(z_ref, o_ref):
    npr = o_ref.shape[0] // 2
    for j in range(npr):
        slab = z_ref[j * 128:(j + 1) * 128, :]
        st = _mxu_t(slab)
        o_ref[2 * j:2 * j + 2, :, :] = st.reshape(2, o_ref.shape[1], 128)


def _unpack(z128, B, T, D):
    nb = B // 128
    rows = T // 2 * 128
    return pl.pallas_call(
        _unpack_body,
        grid=(nb,),
        in_specs=[pl.BlockSpec((rows, 128), lambda i: (i, 0))],
        out_specs=pl.BlockSpec((T, D, 128), lambda i: (0, 0, i)),
        out_shape=jax.ShapeDtypeStruct((T, D, B), jnp.float32),
    )(z128)


def _mask_body(ids_ref, m_ref):
    m_ref[...] = ids_ref[...] != 0


def kernel(ids, table):
    B, T = ids.shape
    V, D = table.shape

    info = plsc.get_sparse_core_info()
    NC, NS = info.num_cores, info.num_subcores

    packed = _pack_table(table.T)
    table_lin = packed.reshape(V, D)

    gather = _make_gather(B, T, V, D, NC, NS)
    z2 = gather(ids.reshape(B * T), table_lin)
    out3 = _unpack(z2.reshape(B * T // 2, 128), B, T, D)
    tokens = jnp.transpose(out3, (2, 0, 1))

    mask = pl.pallas_call(
        _mask_body,
        out_shape=jax.ShapeDtypeStruct((B, T), jnp.bool_),
    )(ids)
    return tokens, mask
